# Initial kernel scaffold; baseline (speedup 1.0000x reference)
import jax
import jax.numpy as jnp
from jax.experimental import pallas as pl


def kernel(W_in, indices):
    raise NotImplementedError("write your pallas kernel here")


# speedup vs baseline: 1.7977x; 1.7977x over previous
import functools

import jax
import jax.numpy as jnp
from jax import lax
from jax.experimental import pallas as pl
from jax.experimental.pallas import tpu as pltpu
from jax.experimental.pallas import tpu_sc as plsc

NC = 2
NS = 16
NW = NC * NS
CHUNK = 128
K = 4
BLOCK = K * CHUNK


def _emb_lookup(n_total, d):
    n_per_w = n_total // NW
    n_blocks = n_per_w // BLOCK

    mesh = plsc.VectorSubcoreMesh(
        core_axis_name="c", subcore_axis_name="s",
        num_cores=NC, num_subcores=NS,
    )

    @functools.partial(
        pl.kernel,
        out_type=jax.ShapeDtypeStruct((n_total, d), jnp.float32),
        mesh=mesh,
        scratch_types=[
            pltpu.VMEM((K, CHUNK), jnp.int32),
            pltpu.VMEM((BLOCK, d), jnp.float32),
            pltpu.SemaphoreType.DMA,
        ],
        compiler_params=pltpu.CompilerParams(use_tc_tiling_on_sc=False),
    )
    def emb(table_hbm, idx_hbm, out_hbm, idx_v, rows_v, sem):
        wid = lax.axis_index("s") * NC + lax.axis_index("c")
        row0 = wid * (n_per_w // CHUNK)

        def step(g, carry):
            r = row0 + g * K
            pltpu.sync_copy(idx_hbm.at[pl.ds(r, K)], idx_v)
            copies = [
                pltpu.async_copy(
                    table_hbm.at[idx_v.at[j]],
                    rows_v.at[pl.ds(j * CHUNK, CHUNK)],
                    sem,
                )
                for j in range(K)
            ]
            for c in copies:
                c.wait()
            pltpu.sync_copy(rows_v, out_hbm.at[pl.ds(r * CHUNK, BLOCK)])
            return carry

        lax.fori_loop(0, n_blocks, step, 0)

    return emb


def kernel(W_in, indices):
    b, l = indices.shape
    d = W_in.shape[1]
    n_total = b * l
    idx2d = indices.reshape(n_total // CHUNK, CHUNK)
    out = _emb_lookup(n_total, d)(W_in, idx2d)
    return out.reshape(b, l, d)
